# baseline (speedup 1.0000x reference)
import functools

import jax
import jax.numpy as jnp
from jax import lax
from jax.experimental import pallas as pl
from jax.experimental.pallas import tpu as pltpu
from jax.experimental.pallas import tpu_sc as plsc

EMB = 2048
NEXP = 64


_TOK_BLK = 1024


def _tc_body(x_ref, w_ref, b_ref, probs_ref, loadp_ref):
    logits = lax.dot_general(
        w_ref[...], x_ref[...], (((1,), (1,)), ((), ())),
        preferred_element_type=jnp.float32)
    logits = logits + b_ref[...]
    m = jnp.max(logits, axis=0, keepdims=True)
    e = jnp.exp(logits - m)
    s = jnp.sum(e, axis=0, keepdims=True)
    p = e / s
    probs_ref[...] = p

    @pl.when(pl.program_id(0) == 0)
    def _init():
        loadp_ref[...] = jnp.zeros_like(loadp_ref)

    loadp_ref[...] += jnp.sum(p, axis=1, keepdims=True)


def _tc_probs(xf, W, b2, chunk_start, chunk_len):
    nblk = chunk_len // _TOK_BLK
    blk0 = chunk_start // _TOK_BLK
    return pl.pallas_call(
        _tc_body,
        grid=(nblk,),
        in_specs=[
            pl.BlockSpec((_TOK_BLK, EMB), lambda i: (blk0 + i, 0)),
            pl.BlockSpec((NEXP, EMB), lambda i: (0, 0)),
            pl.BlockSpec((NEXP, 1), lambda i: (0, 0)),
        ],
        out_specs=[
            pl.BlockSpec((NEXP, _TOK_BLK), lambda i: (0, i)),
            pl.BlockSpec((NEXP, 128), lambda i: (0, 0)),
        ],
        out_shape=[
            jax.ShapeDtypeStruct((NEXP, chunk_len), jnp.float32),
            jax.ShapeDtypeStruct((NEXP, 128), jnp.float32),
        ],
    )(xf, W, b2)


_LANES = 16


def _tourney2(m1s, i1s, m2s, i2s):
    while len(m1s) > 1:
        nm1, ni1, nm2, ni2 = [], [], [], []
        for a in range(0, len(m1s), 2):
            am1, ai1, am2, ai2 = m1s[a], i1s[a], m2s[a], i2s[a]
            bm1, bi1, bm2, bi2 = m1s[a + 1], i1s[a + 1], m2s[a + 1], i2s[a + 1]
            take1 = bm1 > am1
            nm1.append(jnp.where(take1, bm1, am1))
            ni1.append(jnp.where(take1, bi1, ai1))
            ca = jnp.where(take1, am1, am2)
            cai = jnp.where(take1, ai1, ai2)
            cb = jnp.where(take1, bm2, bm1)
            cbi = jnp.where(take1, bi2, bi1)
            take2 = cb > ca
            nm2.append(jnp.where(take2, cb, ca))
            ni2.append(jnp.where(take2, cbi, cai))
        m1s, i1s, m2s, i2s = nm1, ni1, nm2, ni2
    return m1s[0], i1s[0], m2s[0], i2s[0]


_SC_CORES = 2


def _sc_topk_call(probsT, n):
    nworker = 16 * _SC_CORES
    chunk = n // nworker
    groups = chunk // _LANES
    mesh = plsc.VectorSubcoreMesh(
        core_axis_name="c", subcore_axis_name="s", num_cores=_SC_CORES)

    @functools.partial(
        pl.kernel,
        out_type=[
            jax.ShapeDtypeStruct((n,), jnp.float32),
            jax.ShapeDtypeStruct((n,), jnp.float32),
            jax.ShapeDtypeStruct((n,), jnp.int32),
            jax.ShapeDtypeStruct((n,), jnp.int32),
        ],
        mesh=mesh,
        scratch_types=[
            pltpu.VMEM((NEXP, chunk), jnp.float32),
            pltpu.VMEM((chunk,), jnp.float32),
            pltpu.VMEM((chunk,), jnp.float32),
            pltpu.VMEM((chunk,), jnp.int32),
            pltpu.VMEM((chunk,), jnp.int32),
        ],
    )
    def sc_topk(probs_hbm, g1_hbm, g2_hbm, i1_hbm, i2_hbm,
                pv, g1v, g2v, i1v, i2v):
        wid = lax.axis_index("s") * _SC_CORES + lax.axis_index("c")
        base = wid * chunk
        pltpu.sync_copy(probs_hbm.at[:, pl.ds(base, chunk)], pv)

        def group(g, carry):
            off = g * _LANES
            m1s, i1s, m2s, i2s = [], [], [], []
            for j in range(0, NEXP, 2):
                va = pv[j, pl.ds(off, _LANES)]
                vb = pv[j + 1, pl.ds(off, _LANES)]
                take = vb > va
                m1s.append(jnp.where(take, vb, va))
                i1s.append(jnp.where(take, j + 1, j).astype(jnp.int32))
                m2s.append(jnp.where(take, va, vb))
                i2s.append(jnp.where(take, j, j + 1).astype(jnp.int32))
            m1, i1, m2, i2 = _tourney2(m1s, i1s, m2s, i2s)
            denom = m1 + m2 + jnp.float32(1e-8)
            g1v[pl.ds(off, _LANES)] = m1 / denom
            g2v[pl.ds(off, _LANES)] = m2 / denom
            i1v[pl.ds(off, _LANES)] = i1
            i2v[pl.ds(off, _LANES)] = i2
            return carry

        lax.fori_loop(0, groups, group, 0)
        pltpu.sync_copy(g1v, g1_hbm.at[pl.ds(base, chunk)])
        pltpu.sync_copy(g2v, g2_hbm.at[pl.ds(base, chunk)])
        pltpu.sync_copy(i1v, i1_hbm.at[pl.ds(base, chunk)])
        pltpu.sync_copy(i2v, i2_hbm.at[pl.ds(base, chunk)])

    return sc_topk(probsT)


_NCHUNK = 1


def kernel(x, W, b):
    bsz, seq, _ = x.shape
    n = bsz * seq
    cn = n // _NCHUNK
    xf = x.reshape(n, EMB)
    b2 = b.reshape(NEXP, 1)
    g1s, g2s, i1s, i2s, loads = [], [], [], [], []
    for c in range(_NCHUNK):
        probsT, loadp = _tc_probs(xf, W, b2, c * cn, cn)
        g1, g2, i1, i2 = _sc_topk_call(probsT, cn)
        g1s.append(g1)
        g2s.append(g2)
        i1s.append(i1)
        i2s.append(i2)
        loads.append(loadp)
    gates = jnp.stack(
        [jnp.concatenate(g1s), jnp.concatenate(g2s)], axis=-1
    ).reshape(bsz, seq, 2)
    indices = jnp.stack(
        [jnp.concatenate(i1s), jnp.concatenate(i2s)], axis=-1
    ).reshape(bsz, seq, 2)
    load = sum(loads)[:, 0] / jnp.float32(n)
    return gates, indices, load


# speedup vs baseline: 1.0064x; 1.0064x over previous
import functools

import jax
import jax.numpy as jnp
from jax import lax
from jax.experimental import pallas as pl
from jax.experimental.pallas import tpu as pltpu
from jax.experimental.pallas import tpu_sc as plsc

EMB = 2048
NEXP = 64


_TOK_BLK = 1024


def _tc_body(nslab, slab, x_ref, w_ref, b_ref, probs_ref, loadp_ref):
    logits = lax.dot_general(
        w_ref[...], x_ref[...], (((1,), (1,)), ((), ())),
        preferred_element_type=jnp.float32)
    logits = logits + b_ref[...]
    m = jnp.max(logits, axis=0, keepdims=True)
    e = jnp.exp(logits - m)
    s = jnp.sum(e, axis=0, keepdims=True)
    p = e / s
    for k in range(nslab):
        probs_ref[k] = p[:, k * slab:(k + 1) * slab]

    @pl.when(pl.program_id(0) == 0)
    def _init():
        loadp_ref[...] = jnp.zeros_like(loadp_ref)

    loadp_ref[...] += jnp.sum(p, axis=1, keepdims=True)


def _tc_probs(xf, W, b2, chunk_len, slab):
    nblk = chunk_len // _TOK_BLK
    nslab = _TOK_BLK // slab
    return pl.pallas_call(
        functools.partial(_tc_body, nslab, slab),
        grid=(nblk,),
        in_specs=[
            pl.BlockSpec((_TOK_BLK, EMB), lambda i: (i, 0)),
            pl.BlockSpec((NEXP, EMB), lambda i: (0, 0)),
            pl.BlockSpec((NEXP, 1), lambda i: (0, 0)),
        ],
        out_specs=[
            pl.BlockSpec((nslab, NEXP, slab), lambda i: (i, 0, 0)),
            pl.BlockSpec((NEXP, 128), lambda i: (0, 0)),
        ],
        out_shape=[
            jax.ShapeDtypeStruct((chunk_len // slab, NEXP, slab), jnp.float32),
            jax.ShapeDtypeStruct((NEXP, 128), jnp.float32),
        ],
    )(xf, W, b2)


_LANES = 16


def _tourney2(m1s, i1s, m2s, i2s):
    while len(m1s) > 1:
        nm1, ni1, nm2, ni2 = [], [], [], []
        for a in range(0, len(m1s), 2):
            am1, ai1, am2, ai2 = m1s[a], i1s[a], m2s[a], i2s[a]
            bm1, bi1, bm2, bi2 = m1s[a + 1], i1s[a + 1], m2s[a + 1], i2s[a + 1]
            take1 = bm1 > am1
            nm1.append(jnp.where(take1, bm1, am1))
            ni1.append(jnp.where(take1, bi1, ai1))
            ca = jnp.where(take1, am1, am2)
            cai = jnp.where(take1, ai1, ai2)
            cb = jnp.where(take1, bm2, bm1)
            cbi = jnp.where(take1, bi2, bi1)
            take2 = cb > ca
            nm2.append(jnp.where(take2, cb, ca))
            ni2.append(jnp.where(take2, cbi, cai))
        m1s, i1s, m2s, i2s = nm1, ni1, nm2, ni2
    return m1s[0], i1s[0], m2s[0], i2s[0]


_SC_CORES = 2


def _sc_topk_call(probsT, n):
    nworker = 16 * _SC_CORES
    chunk = n // nworker
    groups = chunk // _LANES
    mesh = plsc.VectorSubcoreMesh(
        core_axis_name="c", subcore_axis_name="s", num_cores=_SC_CORES)

    @functools.partial(
        pl.kernel,
        out_type=[
            jax.ShapeDtypeStruct((n,), jnp.float32),
            jax.ShapeDtypeStruct((n,), jnp.float32),
            jax.ShapeDtypeStruct((n,), jnp.int32),
            jax.ShapeDtypeStruct((n,), jnp.int32),
        ],
        mesh=mesh,
        scratch_types=[
            pltpu.VMEM((NEXP, chunk), jnp.float32),
            pltpu.VMEM((chunk,), jnp.float32),
            pltpu.VMEM((chunk,), jnp.float32),
            pltpu.VMEM((chunk,), jnp.int32),
            pltpu.VMEM((chunk,), jnp.int32),
        ],
    )
    def sc_topk(probs_hbm, g1_hbm, g2_hbm, i1_hbm, i2_hbm,
                pv, g1v, g2v, i1v, i2v):
        wid = lax.axis_index("s") * _SC_CORES + lax.axis_index("c")
        base = wid * chunk
        pltpu.sync_copy(probs_hbm.at[wid], pv)

        def group(g, carry):
            off = g * _LANES
            m1s, i1s, m2s, i2s = [], [], [], []
            for j in range(0, NEXP, 2):
                va = pv[j, pl.ds(off, _LANES)]
                vb = pv[j + 1, pl.ds(off, _LANES)]
                take = vb > va
                m1s.append(jnp.where(take, vb, va))
                i1s.append(jnp.where(take, j + 1, j).astype(jnp.int32))
                m2s.append(jnp.where(take, va, vb))
                i2s.append(jnp.where(take, j, j + 1).astype(jnp.int32))
            m1, i1, m2, i2 = _tourney2(m1s, i1s, m2s, i2s)
            denom = m1 + m2 + jnp.float32(1e-8)
            g1v[pl.ds(off, _LANES)] = m1 / denom
            g2v[pl.ds(off, _LANES)] = m2 / denom
            i1v[pl.ds(off, _LANES)] = i1
            i2v[pl.ds(off, _LANES)] = i2
            return carry

        lax.fori_loop(0, groups, group, 0)
        pltpu.sync_copy(g1v, g1_hbm.at[pl.ds(base, chunk)])
        pltpu.sync_copy(g2v, g2_hbm.at[pl.ds(base, chunk)])
        pltpu.sync_copy(i1v, i1_hbm.at[pl.ds(base, chunk)])
        pltpu.sync_copy(i2v, i2_hbm.at[pl.ds(base, chunk)])

    return sc_topk(probsT)


def kernel(x, W, b):
    bsz, seq, _ = x.shape
    n = bsz * seq
    sc_chunk = n // (16 * _SC_CORES)
    xf = x.reshape(n, EMB)
    b2 = b.reshape(NEXP, 1)
    probsT, loadp = _tc_probs(xf, W, b2, n, sc_chunk)
    g1, g2, i1, i2 = _sc_topk_call(probsT, n)
    gates = jnp.stack([g1, g2], axis=-1).reshape(bsz, seq, 2)
    indices = jnp.stack([i1, i2], axis=-1).reshape(bsz, seq, 2)
    load = loadp[:, 0] / jnp.float32(n)
    return gates, indices, load
